# baseline (speedup 1.0000x reference)
import functools

import jax
import jax.numpy as jnp
from jax import lax
from jax.experimental import pallas as pl
from jax.experimental.pallas import tpu as pltpu
from jax.experimental.pallas import tpu_sc as plsc

N = 10000
E = 320000
D = 128

_NC = 2
_NS = 16
_NW = _NC * _NS
_C = 128
_RPW = -(-E // (_NW * _C) // 8) * 8
_EROWS = _NW * _RPW
_EPAD = _EROWS * _C
_NPAD = N + _C
_NDEG = N + _C

_RB = 640
_RB_LAST = N - (_NS - 1) * _RB

_mesh = plsc.VectorSubcoreMesh(core_axis_name="c", subcore_axis_name="s")


def _stripe_copy(copy_fn, total=N):
    s = lax.axis_index("s")

    @pl.when(s < _NS - 1)
    def _():
        copy_fn(s * _RB, _RB)

    @pl.when(s == _NS - 1)
    def _():
        copy_fn((_NS - 1) * _RB, total - (_NS - 1) * _RB)


_DEG_SCRATCH = [
    pltpu.VMEM((_RPW, _C), jnp.int32),
    pltpu.VMEM((_C,), jnp.float32),
    pltpu.VMEM((_RB,), jnp.float32),
    pltpu.VMEM_SHARED((_NDEG,), jnp.float32),
    pltpu.SemaphoreType.DMA,
]


def _deg_body(dst_hbm, deg_out, dst_all, ones_v, stage_v, deg_sh, sem_i):
    c = lax.axis_index("c")
    s = lax.axis_index("s")
    wid = c * _NS + s
    idx_cp = pltpu.async_copy(dst_hbm.at[pl.ds(wid * _RPW, _RPW)], dst_all,
                              sem_i)
    for i in range(_C // 16):
        ones_v[pl.ds(i * 16, 16)] = jnp.full((16,), 1.0, jnp.float32)

    def zero_stage(k, carry):
        stage_v[pl.ds(k * 16, 16)] = jnp.zeros((16,), jnp.float32)
        return carry

    lax.fori_loop(0, _RB // 16, zero_stage, 0)
    _stripe_copy(lambda off, sz: pltpu.sync_copy(
        stage_v.at[pl.ds(0, sz)], deg_sh.at[pl.ds(off, sz)]), total=_NDEG)
    idx_cp.wait()
    plsc.subcore_barrier()

    def body(k, carry):
        pltpu.sync_copy(ones_v, deg_sh.at[dst_all.at[k]], add=True)
        return carry

    lax.fori_loop(0, _RPW, body, 0)
    plsc.subcore_barrier()

    def wb(off, sz):
        pltpu.sync_copy(deg_sh.at[pl.ds(off, sz)], stage_v.at[pl.ds(0, sz)])
        pltpu.sync_copy(stage_v.at[pl.ds(0, sz)],
                        deg_out.at[pl.ds(c * N + off, sz)])

    _stripe_copy(wb)


_HOP_SCRATCH = [
    pltpu.VMEM((_RPW, _C), jnp.int32),
    pltpu.VMEM((_C,), jnp.int32),
    pltpu.VMEM((_C,), jnp.int32),
    pltpu.VMEM((_C, D), jnp.float32),
    pltpu.VMEM((_C, D), jnp.float32),
    pltpu.VMEM_SHARED((_NPAD, D), jnp.float32),
    pltpu.SemaphoreType.DMA,
    pltpu.SemaphoreType.DMA,
    pltpu.SemaphoreType.DMA,
    pltpu.SemaphoreType.DMA,
    pltpu.SemaphoreType.DMA,
]


def _hop_body(x_hbm, src_hbm, dst_hbm, zeros_hbm, out_hbm, src_all, dst_a,
              dst_b, rows_a, rows_b, acc_sh, sem_a, sem_b, sem_da, sem_db,
              sem_i):
    c = lax.axis_index("c")
    s = lax.axis_index("s")
    wid = c * _NS + s
    base0 = wid * _RPW
    src_cp = pltpu.async_copy(src_hbm.at[pl.ds(base0, _RPW)], src_all, sem_i)
    _stripe_copy(lambda off, sz: pltpu.sync_copy(
        zeros_hbm.at[pl.ds(off, sz)], acc_sh.at[pl.ds(off, sz)]))
    src_cp.wait()
    plsc.subcore_barrier()

    def start_dst(k, dst_v, sem):
        pltpu.async_copy(dst_hbm.at[base0 + k], dst_v, sem)

    def wait_dst(dst_v, sem):
        pltpu.make_async_copy(dst_hbm.at[base0], dst_v, sem).wait()

    def start_gather(k, rows_v, sem):
        pltpu.async_copy(x_hbm.at[src_all.at[k]], rows_v, sem)

    def wait_gather(rows_v, sem):
        pltpu.make_async_copy(x_hbm.at[src_all.at[0]], rows_v, sem).wait()

    start_dst(0, dst_a, sem_da)
    start_gather(0, rows_a, sem_a)

    def body(j, carry):
        k1 = 2 * j + 1
        k2 = 2 * j + 2
        start_dst(k1, dst_b, sem_db)
        start_gather(k1, rows_b, sem_b)
        wait_gather(rows_a, sem_a)
        wait_dst(dst_a, sem_da)
        pltpu.sync_copy(rows_a, acc_sh.at[dst_a], add=True)

        @pl.when(k2 < _RPW)
        def _():
            start_dst(k2, dst_a, sem_da)
            start_gather(k2, rows_a, sem_a)

        wait_gather(rows_b, sem_b)
        wait_dst(dst_b, sem_db)
        pltpu.sync_copy(rows_b, acc_sh.at[dst_b], add=True)
        return carry

    lax.fori_loop(0, _RPW // 2, body, 0)
    plsc.subcore_barrier()
    _stripe_copy(lambda off, sz: pltpu.sync_copy(
        acc_sh.at[pl.ds(off, sz)], out_hbm.at[c, pl.ds(off, sz)]))


_deg_kernel = functools.partial(
    pl.kernel,
    out_type=jax.ShapeDtypeStruct((_NC * N,), jnp.float32),
    mesh=_mesh,
    scratch_types=_DEG_SCRATCH,
)(_deg_body)

_hop_kernel = functools.partial(
    pl.kernel,
    out_type=jax.ShapeDtypeStruct((_NC, N, D), jnp.float32),
    mesh=_mesh,
    scratch_types=_HOP_SCRATCH,
)(_hop_body)


_RROWS = 2000


def _prescale_body(deg_ref, feat_ref, o_ref):
    deg = jnp.maximum(deg_ref[0] + deg_ref[1], 1.0)
    o_ref[...] = feat_ref[...] * lax.rsqrt(deg)


def _combine_body(recip, deg_ref, p_ref, o_ref):
    deg = jnp.maximum(deg_ref[0] + deg_ref[1], 1.0)
    scale = jnp.where(recip, 1.0 / deg, lax.rsqrt(deg))
    o_ref[...] = (p_ref[0] + p_ref[1]) * scale


def _prescale(deg2, feat):
    return pl.pallas_call(
        _prescale_body,
        grid=(N // _RROWS,),
        in_specs=[
            pl.BlockSpec((_NC, _RROWS, 1), lambda i: (0, i, 0)),
            pl.BlockSpec((_RROWS, D), lambda i: (i, 0)),
        ],
        out_specs=pl.BlockSpec((_RROWS, D), lambda i: (i, 0)),
        out_shape=jax.ShapeDtypeStruct((N, D), jnp.float32),
    )(deg2, feat)


def _combine(deg2, parts, recip):
    return pl.pallas_call(
        functools.partial(_combine_body, recip),
        grid=(N // _RROWS,),
        in_specs=[
            pl.BlockSpec((_NC, _RROWS, 1), lambda i: (0, i, 0)),
            pl.BlockSpec((_NC, _RROWS, D), lambda i: (0, i, 0)),
        ],
        out_specs=pl.BlockSpec((_RROWS, D), lambda i: (i, 0)),
        out_shape=jax.ShapeDtypeStruct((N, D), jnp.float32),
    )(deg2, parts)


def kernel(feat, edge_index):
    src = edge_index[0]
    dst = edge_index[1]
    pad = _EPAD - E
    src2d = jnp.concatenate(
        [src, jnp.zeros((pad,), jnp.int32)]).reshape(_EROWS, _C)
    dst2d = jnp.concatenate(
        [dst, N + (jnp.arange(pad, dtype=jnp.int32) % _C)]).reshape(
            _EROWS, _C)
    zeros_nd = jnp.zeros((N, D), jnp.float32)

    deg_part = _deg_kernel(dst2d)
    deg2 = deg_part.reshape(_NC, N, 1)

    x0 = _prescale(deg2, feat)
    p1 = _hop_kernel(x0, src2d, dst2d, zeros_nd)
    x1 = _combine(deg2, p1, recip=True)
    p2 = _hop_kernel(x1, src2d, dst2d, zeros_nd)
    return _combine(deg2, p2, recip=False)


# speedup vs baseline: 3.6562x; 3.6562x over previous
import functools

import jax
import jax.numpy as jnp
from jax import lax
from jax.experimental import pallas as pl
from jax.experimental.pallas import tpu as pltpu
from jax.experimental.pallas import tpu_sc as plsc

N = 10000
E = 320000
D = 128

_NC = 2
_NS = 16
_NW = _NC * _NS
_C = 128
_RPW = -(-E // (_NW * _C) // 8) * 8
_EROWS = _NW * _RPW
_EPAD = _EROWS * _C
_NPAD = N + _C
_NDEG = N + _C

_RB = 640
_RB_LAST = N - (_NS - 1) * _RB

_mesh = plsc.VectorSubcoreMesh(core_axis_name="c", subcore_axis_name="s")


def _stripe_copy(copy_fn, total=N):
    s = lax.axis_index("s")

    @pl.when(s < _NS - 1)
    def _():
        copy_fn(s * _RB, _RB)

    @pl.when(s == _NS - 1)
    def _():
        copy_fn((_NS - 1) * _RB, total - (_NS - 1) * _RB)


_DEG_SCRATCH = [
    pltpu.VMEM((_RPW, _C), jnp.int32),
    pltpu.VMEM((_C,), jnp.float32),
    pltpu.VMEM((_RB,), jnp.float32),
    pltpu.VMEM_SHARED((_NDEG,), jnp.float32),
    pltpu.SemaphoreType.DMA,
    pltpu.SemaphoreType.DMA,
]


def _deg_body(dst_hbm, deg_out, dst_all, ones_v, stage_v, deg_sh, sem_i,
              sem_s):
    c = lax.axis_index("c")
    s = lax.axis_index("s")
    wid = c * _NS + s
    idx_cp = pltpu.async_copy(dst_hbm.at[pl.ds(wid * _RPW, _RPW)], dst_all,
                              sem_i)
    for i in range(_C // 16):
        ones_v[pl.ds(i * 16, 16)] = jnp.full((16,), 1.0, jnp.float32)

    def zero_stage(k, carry):
        stage_v[pl.ds(k * 16, 16)] = jnp.zeros((16,), jnp.float32)
        return carry

    lax.fori_loop(0, _RB // 16, zero_stage, 0)
    _stripe_copy(lambda off, sz: pltpu.sync_copy(
        stage_v.at[pl.ds(0, sz)], deg_sh.at[pl.ds(off, sz)]), total=_NDEG)
    idx_cp.wait()
    plsc.subcore_barrier()

    def body(k, carry):
        pltpu.async_copy(ones_v, deg_sh.at[dst_all.at[k]], sem_s, add=True)
        return carry

    lax.fori_loop(0, _RPW, body, 0)

    def drain(k, carry):
        pltpu.make_async_copy(ones_v, deg_sh.at[dst_all.at[0]], sem_s).wait()
        return carry

    lax.fori_loop(0, _RPW, drain, 0)
    plsc.subcore_barrier()

    def wb(off, sz):
        pltpu.sync_copy(deg_sh.at[pl.ds(off, sz)], stage_v.at[pl.ds(0, sz)])
        pltpu.sync_copy(stage_v.at[pl.ds(0, sz)],
                        deg_out.at[pl.ds(c * N + off, sz)])

    _stripe_copy(wb)


_HOP_SCRATCH = [
    pltpu.VMEM((_RPW, _C), jnp.int32),
    pltpu.VMEM((_C,), jnp.int32),
    pltpu.VMEM((_C,), jnp.int32),
    pltpu.VMEM((_C, D), jnp.float32),
    pltpu.VMEM((_C, D), jnp.float32),
    pltpu.VMEM_SHARED((_NPAD, D), jnp.float32),
    pltpu.SemaphoreType.DMA,
    pltpu.SemaphoreType.DMA,
    pltpu.SemaphoreType.DMA,
    pltpu.SemaphoreType.DMA,
    pltpu.SemaphoreType.DMA,
]


def _hop_body(x_hbm, src_hbm, dst_hbm, zeros_hbm, out_hbm, src_all, dst_a,
              dst_b, rows_a, rows_b, acc_sh, sem_a, sem_b, sem_da, sem_db,
              sem_i):
    c = lax.axis_index("c")
    s = lax.axis_index("s")
    wid = c * _NS + s
    base0 = wid * _RPW
    src_cp = pltpu.async_copy(src_hbm.at[pl.ds(base0, _RPW)], src_all, sem_i)
    _stripe_copy(lambda off, sz: pltpu.sync_copy(
        zeros_hbm.at[pl.ds(off, sz)], acc_sh.at[pl.ds(off, sz)]))
    src_cp.wait()
    plsc.subcore_barrier()

    def start_dst(k, dst_v, sem):
        pltpu.async_copy(dst_hbm.at[base0 + k], dst_v, sem)

    def wait_dst(dst_v, sem):
        pltpu.make_async_copy(dst_hbm.at[base0], dst_v, sem).wait()

    def start_gather(k, rows_v, sem):
        pltpu.async_copy(x_hbm.at[src_all.at[k]], rows_v, sem)

    def wait_gather(rows_v, sem):
        pltpu.make_async_copy(x_hbm.at[src_all.at[0]], rows_v, sem).wait()

    start_dst(0, dst_a, sem_da)
    start_gather(0, rows_a, sem_a)

    def body(j, carry):
        k1 = 2 * j + 1
        k2 = 2 * j + 2
        start_dst(k1, dst_b, sem_db)
        start_gather(k1, rows_b, sem_b)
        wait_gather(rows_a, sem_a)
        wait_dst(dst_a, sem_da)
        pltpu.sync_copy(rows_a, acc_sh.at[dst_a], add=True)

        @pl.when(k2 < _RPW)
        def _():
            start_dst(k2, dst_a, sem_da)
            start_gather(k2, rows_a, sem_a)

        wait_gather(rows_b, sem_b)
        wait_dst(dst_b, sem_db)
        pltpu.sync_copy(rows_b, acc_sh.at[dst_b], add=True)
        return carry

    lax.fori_loop(0, _RPW // 2, body, 0)
    plsc.subcore_barrier()
    _stripe_copy(lambda off, sz: pltpu.sync_copy(
        acc_sh.at[pl.ds(off, sz)], out_hbm.at[c, pl.ds(off, sz)]))


_deg_kernel = functools.partial(
    pl.kernel,
    out_type=jax.ShapeDtypeStruct((_NC * N,), jnp.float32),
    mesh=_mesh,
    scratch_types=_DEG_SCRATCH,
)(_deg_body)

_hop_kernel = functools.partial(
    pl.kernel,
    out_type=jax.ShapeDtypeStruct((_NC, N, D), jnp.float32),
    mesh=_mesh,
    scratch_types=_HOP_SCRATCH,
)(_hop_body)


_RROWS = 2000


def _prescale_body(deg_ref, feat_ref, o_ref):
    deg = jnp.maximum(deg_ref[0] + deg_ref[1], 1.0)
    o_ref[...] = feat_ref[...] * lax.rsqrt(deg)


def _combine_body(recip, deg_ref, p_ref, o_ref):
    deg = jnp.maximum(deg_ref[0] + deg_ref[1], 1.0)
    scale = jnp.where(recip, 1.0 / deg, lax.rsqrt(deg))
    o_ref[...] = (p_ref[0] + p_ref[1]) * scale


def _prescale(deg2, feat):
    return pl.pallas_call(
        _prescale_body,
        grid=(N // _RROWS,),
        in_specs=[
            pl.BlockSpec((_NC, _RROWS, 1), lambda i: (0, i, 0)),
            pl.BlockSpec((_RROWS, D), lambda i: (i, 0)),
        ],
        out_specs=pl.BlockSpec((_RROWS, D), lambda i: (i, 0)),
        out_shape=jax.ShapeDtypeStruct((N, D), jnp.float32),
    )(deg2, feat)


def _combine(deg2, parts, recip):
    return pl.pallas_call(
        functools.partial(_combine_body, recip),
        grid=(N // _RROWS,),
        in_specs=[
            pl.BlockSpec((_NC, _RROWS, 1), lambda i: (0, i, 0)),
            pl.BlockSpec((_NC, _RROWS, D), lambda i: (0, i, 0)),
        ],
        out_specs=pl.BlockSpec((_RROWS, D), lambda i: (i, 0)),
        out_shape=jax.ShapeDtypeStruct((N, D), jnp.float32),
    )(deg2, parts)


def kernel(feat, edge_index):
    src = edge_index[0]
    dst = edge_index[1]
    pad = _EPAD - E
    src2d = jnp.concatenate(
        [src, (jnp.arange(pad, dtype=jnp.int32) * 73) % N]).reshape(
            _EROWS, _C)
    dst2d = jnp.concatenate(
        [dst, N + (jnp.arange(pad, dtype=jnp.int32) % _C)]).reshape(
            _EROWS, _C)
    zeros_nd = jnp.zeros((N, D), jnp.float32)

    deg_part = _deg_kernel(dst2d)
    deg2 = deg_part.reshape(_NC, N, 1)

    x0 = _prescale(deg2, feat)
    p1 = _hop_kernel(x0, src2d, dst2d, zeros_nd)
    x1 = _combine(deg2, p1, recip=True)
    p2 = _hop_kernel(x1, src2d, dst2d, zeros_nd)
    return _combine(deg2, p2, recip=False)
